# baseline (speedup 1.0000x reference)
import functools

import jax
import jax.numpy as jnp
from jax import lax
from jax.experimental import pallas as pl
from jax.experimental.pallas import tpu as pltpu
from jax.experimental.pallas import tpu_sc as plsc

_EPS = 1e-20
_R = 128
_C = 100000
_B = 8192
_NB = (_C + _B - 1) // _B

_BIG_F32 = 1e9

_NC = 2
_NS = 16
_ROWS_PER_TILE = _R // (_NC * _NS)
_CHUNK = 20000
_FILL_UNROLL = 10


def _argmax_body(x_ref, u_ref, idx_out, max_ref):
    j = pl.program_id(0)
    col0 = (j * _B).astype(jnp.float32)
    iota_f = lax.broadcasted_iota(jnp.int32, (_R, _B), 1).astype(jnp.float32)
    gcol = col0 + iota_f

    t = -jnp.log(u_ref[...] + _EPS) + _EPS
    f = jnp.exp(x_ref[...]) / t
    f = jnp.where(gcol < float(_C), f, -1.0)
    m = jnp.max(f, axis=1, keepdims=True)
    cand = jnp.min(jnp.where(f == m, gcol, _BIG_F32), axis=1, keepdims=True)

    @pl.when(j == 0)
    def _init():
        max_ref[...] = m
        idx_out[...] = cand.astype(jnp.int32)

    @pl.when(j > 0)
    def _acc():
        better = m > max_ref[...]
        max_ref[...] = jnp.where(better, m, max_ref[...])
        idx_out[...] = jnp.where(better, cand.astype(jnp.int32), idx_out[...])


def _tc_argmax(x, U):
    return pl.pallas_call(
        _argmax_body,
        grid=(_NB,),
        in_specs=[
            pl.BlockSpec((_R, _B), lambda j: (0, j)),
            pl.BlockSpec((_R, _B), lambda j: (0, j)),
        ],
        out_specs=pl.BlockSpec((_R, 1), lambda j: (0, 0)),
        out_shape=jax.ShapeDtypeStruct((_R, 1), jnp.int32),
        scratch_shapes=[pltpu.VMEM((_R, 1), jnp.float32)],
        compiler_params=pltpu.CompilerParams(
            dimension_semantics=("arbitrary",),
        ),
    )(x, U)


def _sc_body(idx_hbm, out_hbm, zero_v, idx_v, ones_v, sem, sem2):
    c = lax.axis_index("c")
    s = lax.axis_index("s")

    zeros16 = jnp.zeros((16,), jnp.float32)

    def fill(i, carry):
        base = i * (16 * _FILL_UNROLL)
        for u in range(_FILL_UNROLL):
            zero_v[pl.ds(base + u * 16, 16)] = zeros16
        return carry

    lax.fori_loop(0, _CHUNK // (16 * _FILL_UNROLL), fill, 0)

    row0 = (_NS * c + s) * _ROWS_PER_TILE
    copies = []
    for j in range(_ROWS_PER_TILE):
        for k in range(_C // _CHUNK):
            off = (row0 + j) * _C + k * _CHUNK
            copies.append(
                pltpu.make_async_copy(zero_v, out_hbm.at[pl.ds(off, _CHUNK)], sem)
            )
    for cp in copies:
        cp.start()
    for cp in copies:
        cp.wait()

    plsc.subcore_barrier()

    @pl.when(s < 4)
    def _scatter():
        pltpu.sync_copy(idx_hbm, idx_v)
        ones_v[...] = jnp.ones((16,), jnp.float32)
        r0 = (_NS * c + s * 4) * _ROWS_PER_TILE
        idx16 = idx_v[pl.ds(r0, 16)]
        rows = r0 + lax.iota(jnp.int32, 16)
        flat = rows * _C + idx16
        pltpu.async_copy(ones_v, out_hbm.at[flat], sem2).wait()


@functools.lru_cache(maxsize=1)
def _sc_onehot_call():
    return pl.kernel(
        _sc_body,
        out_type=jax.ShapeDtypeStruct((_R * _C,), jnp.float32),
        mesh=plsc.VectorSubcoreMesh(
            core_axis_name="c", subcore_axis_name="s",
            num_cores=_NC, num_subcores=_NS,
        ),
        scratch_types=[
            pltpu.VMEM((_CHUNK,), jnp.float32),
            pltpu.VMEM((_R,), jnp.int32),
            pltpu.VMEM((16,), jnp.float32),
            pltpu.SemaphoreType.DMA,
            pltpu.SemaphoreType.DMA,
        ],
    )


@jax.jit
def kernel(x, U):
    idx = _tc_argmax(x, U)
    flat = _sc_onehot_call()(idx.reshape(_R))
    return flat.reshape(_R, _C)
